# Initial kernel scaffold; baseline (speedup 1.0000x reference)
import jax
import jax.numpy as jnp
from jax.experimental import pallas as pl


def kernel(ids_observation, action_mask, edges, defense_indices, seq_lens, W_embed, b_embed, W1, b1, W2, b2, W_pol, b_pol, W_val, b_val):
    raise NotImplementedError("write your pallas kernel here")


# speedup vs baseline: 33.2559x; 33.2559x over previous
import functools

import jax
import jax.numpy as jnp
import numpy as np
from jax import lax
from jax.experimental import pallas as pl
from jax.experimental.pallas import tpu as pltpu
from jax.experimental.pallas import tpu_sc as plsc

N = 100000
E = 1600000
H = 32
D = 128
FLOAT_MIN = -3.4028235e38
FLOAT_MAX = 3.4028235e38

NC = 2
NS = 16
N2 = N // NC
TRASH = 1200
M = N2 + TRASH
NP = NC * M
PER_TILE = M // NS
WIN = 512
NWIN = 196
E_PAD_T = WIN * NWIN
E_PAD = E_PAD_T * NS

_mesh = plsc.VectorSubcoreMesh(
    core_axis_name="c", subcore_axis_name="s", num_cores=NC, num_subcores=NS
)


def _msgpass_body(with_deg, padded_table, *refs):
    if with_deg:
        (h_hbm, src_hbm, dst_hbm, za_hbm, zd_hbm, agg_hbm, deg_hbm,
         agg_sp, deg_sp, src_v, dst_v, src2d, dst2d, rows, ones_v,
         gsem, ssem) = refs
    else:
        (h_hbm, src_hbm, dst_hbm, za_hbm, agg_hbm,
         agg_sp, src_v, dst_v, src2d, dst2d, rows,
         gsem, ssem) = refs

    c = lax.axis_index("c")
    s = lax.axis_index("s")
    base = c * N2

    pltpu.sync_copy(za_hbm.at[pl.ds(s * PER_TILE, PER_TILE)],
                    agg_sp.at[pl.ds(s * PER_TILE, PER_TILE)])
    if with_deg:
        pltpu.sync_copy(zd_hbm.at[pl.ds(s * PER_TILE, PER_TILE)],
                        deg_sp.at[pl.ds(s * PER_TILE, PER_TILE)])
        one = jnp.full((16,), 1.0, dtype=jnp.float32)
        for j in range(8):
            ones_v[pl.ds(j * 16, 16)] = one
    plsc.subcore_barrier()

    lanes = lax.broadcasted_iota(jnp.int32, (16,), 0)

    def window(w, carry):
        e0 = s * E_PAD_T + w * WIN
        pltpu.sync_copy(src_hbm.at[pl.ds(e0, WIN)], src_v)
        pltpu.sync_copy(dst_hbm.at[pl.ds(e0, WIN)], dst_v)
        for i in range(WIN // 16):
            sv = src_v[pl.ds(i * 16, 16)]
            if padded_table:
                sv = sv + jnp.where(sv >= N2, TRASH, 0)
            src2d[i // 8, pl.ds((i % 8) * 16, 16)] = sv
        gcopies = []
        for j in range(WIN // 128):
            gcopies.append(pltpu.async_copy(
                h_hbm.at[src2d.at[j]], rows.at[pl.ds(j * 128, 128)], gsem))
        for i in range(WIN // 16):
            d = dst_v[pl.ds(i * 16, 16)] - base
            oob = (d < 0) | (d >= N2)
            mix = lanes + (i * 16 + s * 67 + w * 131)
            tr = N2 + (mix & 1023)
            dst2d[i // 8, pl.ds((i % 8) * 16, 16)] = jnp.where(oob, tr, d)
        for cp in gcopies:
            cp.wait()
        scopies = []
        for j in range(WIN // 128):
            scopies.append(pltpu.async_copy(
                rows.at[pl.ds(j * 128, 128)], agg_sp.at[dst2d.at[j]], ssem,
                add=True))
        if with_deg:
            dcopies = [pltpu.async_copy(ones_v, deg_sp.at[dst2d.at[j]], ssem,
                                        add=True) for j in range(WIN // 128)]
            scopies += dcopies
        for cp in scopies:
            cp.wait()
        return carry

    lax.fori_loop(0, NWIN, window, 0)
    plsc.subcore_barrier()

    pltpu.sync_copy(agg_sp.at[pl.ds(s * PER_TILE, PER_TILE)],
                    agg_hbm.at[pl.ds(c * M + s * PER_TILE, PER_TILE)])
    if with_deg:
        pltpu.sync_copy(deg_sp.at[pl.ds(s * PER_TILE, PER_TILE)],
                        deg_hbm.at[c, pl.ds(s * PER_TILE, PER_TILE)])


def _make_msgpass(with_deg, padded_table):
    out_type = [jax.ShapeDtypeStruct((NP, H), jnp.float32)]
    scratch = [pltpu.VMEM_SHARED((M, H), jnp.float32)]
    if with_deg:
        out_type.append(jax.ShapeDtypeStruct((NC, M), jnp.float32))
        scratch.append(pltpu.VMEM_SHARED((M,), jnp.float32))
    scratch += [
        pltpu.VMEM((WIN,), jnp.int32),
        pltpu.VMEM((WIN,), jnp.int32),
        pltpu.VMEM((WIN // 128, 128), jnp.int32),
        pltpu.VMEM((WIN // 128, 128), jnp.int32),
        pltpu.VMEM((WIN, H), jnp.float32),
    ]
    if with_deg:
        scratch.append(pltpu.VMEM((128,), jnp.float32))
    scratch += [pltpu.SemaphoreType.DMA, pltpu.SemaphoreType.DMA]
    return pl.kernel(
        functools.partial(_msgpass_body, with_deg, padded_table),
        out_type=tuple(out_type) if with_deg else out_type[0],
        mesh=_mesh,
        scratch_types=scratch,
        compiler_params=pltpu.CompilerParams(use_tc_tiling_on_sc=False),
    )


_msgpass_deg = _make_msgpass(True, False)
_msgpass = _make_msgpass(False, True)


def _head_body(h_hbm, didx_hbm, emb_hbm, didx_v, didx_m, rows, gsem):
    c = lax.axis_index("c")
    s = lax.axis_index("s")

    @pl.when((c == 0) & (s == 0))
    def _():
        pltpu.sync_copy(didx_hbm, didx_v)
        for i in range(D // 16):
            dv = didx_v[pl.ds(i * 16, 16)]
            didx_m[pl.ds(i * 16, 16)] = dv + jnp.where(dv >= N2, TRASH, 0)
        pltpu.async_copy(h_hbm.at[didx_m], rows, gsem).wait()
        pltpu.sync_copy(rows, emb_hbm)


_head = pl.kernel(
    _head_body,
    out_type=jax.ShapeDtypeStruct((D, H), jnp.float32),
    mesh=_mesh,
    scratch_types=[
        pltpu.VMEM((D,), jnp.int32),
        pltpu.VMEM((D,), jnp.int32),
        pltpu.VMEM((D, H), jnp.float32),
        pltpu.SemaphoreType.DMA,
    ],
    compiler_params=pltpu.CompilerParams(use_tc_tiling_on_sc=False,
                                         needs_layout_passes=False),
)


def _pol_body(emb_ref, wp_ref, bm_ref, o_ref):
    o_ref[...] = (
        jnp.dot(emb_ref[...], wp_ref[...],
                preferred_element_type=jnp.float32) + bm_ref[...])


_pol = pl.pallas_call(
    _pol_body,
    in_specs=[
        pl.BlockSpec((D, H), lambda: (0, 0)),
        pl.BlockSpec((H, 1), lambda: (0, 0)),
        pl.BlockSpec((D, 1), lambda: (0, 0)),
    ],
    out_specs=pl.BlockSpec((D, 1), lambda: (0, 0)),
    out_shape=jax.ShapeDtypeStruct((D, 1), jnp.float32),
)


BN = 3200
NB = NP // BN


def _embed_body(x_ref, w_ref, b_ref, o_ref):
    o_ref[...] = jnp.tanh(x_ref[...] * w_ref[...] + b_ref[...])


_embed = pl.pallas_call(
    _embed_body,
    grid=(N // 4000,),
    in_specs=[
        pl.BlockSpec((4000, 1), lambda i: (i, 0)),
        pl.BlockSpec((1, H), lambda i: (0, 0)),
        pl.BlockSpec((1, H), lambda i: (0, 0)),
    ],
    out_specs=pl.BlockSpec((4000, H), lambda i: (i, 0)),
    out_shape=jax.ShapeDtypeStruct((N, H), jnp.float32),
)


def _dense_body(agg_ref, deg_ref, w_ref, b_ref, o_ref):
    a = agg_ref[...] / jnp.maximum(deg_ref[...], 1.0)
    o_ref[...] = jnp.maximum(
        jnp.dot(a, w_ref[...], preferred_element_type=jnp.float32)
        + b_ref[...], 0.0)


_dense = pl.pallas_call(
    _dense_body,
    grid=(NB,),
    in_specs=[
        pl.BlockSpec((BN, H), lambda i: (i, 0)),
        pl.BlockSpec((BN, 1), lambda i: (i, 0)),
        pl.BlockSpec((H, H), lambda i: (0, 0)),
        pl.BlockSpec((1, H), lambda i: (0, 0)),
    ],
    out_specs=pl.BlockSpec((BN, H), lambda i: (i, 0)),
    out_shape=jax.ShapeDtypeStruct((NP, H), jnp.float32),
)


def _dense_val_body(agg_ref, deg_ref, w_ref, b_ref, wv_ref, bv_ref,
                    h_ref, val_ref, acc_ref):
    i = pl.program_id(0)
    a = agg_ref[...] / jnp.maximum(deg_ref[...], 1.0)
    hb = jnp.maximum(
        jnp.dot(a, w_ref[...], preferred_element_type=jnp.float32)
        + b_ref[...], 0.0)
    h_ref[...] = hb

    @pl.when(i == 0)
    def _():
        acc_ref[...] = jnp.zeros_like(acc_ref)

    half_blocks = NB // NC
    row_in_half = (i % half_blocks) * BN
    limit = jnp.minimum(jnp.maximum(N2 - row_in_half, 0), BN)
    riota = lax.broadcasted_iota(jnp.int32, (BN, 1), 0)
    hmask = jnp.where(riota < limit, hb, 0.0)
    acc_ref[...] += jnp.sum(hmask, axis=0, keepdims=True)

    @pl.when(i == NB - 1)
    def _():
        val_ref[...] = (
            jnp.dot(acc_ref[...], wv_ref[...],
                    preferred_element_type=jnp.float32) / np.float32(N)
            + bv_ref[...])


_dense_val = pl.pallas_call(
    _dense_val_body,
    grid=(NB,),
    in_specs=[
        pl.BlockSpec((BN, H), lambda i: (i, 0)),
        pl.BlockSpec((BN, 1), lambda i: (i, 0)),
        pl.BlockSpec((H, H), lambda i: (0, 0)),
        pl.BlockSpec((1, H), lambda i: (0, 0)),
        pl.BlockSpec((H, 1), lambda i: (0, 0)),
        pl.BlockSpec((1, 1), lambda i: (0, 0)),
    ],
    out_specs=[
        pl.BlockSpec((BN, H), lambda i: (i, 0)),
        pl.BlockSpec((1, 1), lambda i: (0, 0)),
    ],
    out_shape=[
        jax.ShapeDtypeStruct((NP, H), jnp.float32),
        jax.ShapeDtypeStruct((1, 1), jnp.float32),
    ],
    scratch_shapes=[pltpu.VMEM((1, H), jnp.float32)],
)


def kernel(ids_observation, action_mask, edges, defense_indices, seq_lens,
           W_embed, b_embed, W1, b1, W2, b2, W_pol, b_pol, W_val, b_val):
    x = ids_observation[0].reshape(N, 1)
    src = edges[0, :, 0]
    dst = edges[0, :, 1]
    npad = E_PAD - E
    pad_src = (jnp.arange(npad, dtype=jnp.int32) * 977) % N
    pad_dst = jnp.full((npad,), N, dtype=jnp.int32)
    src_p = jnp.concatenate([src, pad_src])
    dst_p = jnp.concatenate([dst, pad_dst])
    za = jnp.zeros((M, H), jnp.float32)
    zd = jnp.zeros((M,), jnp.float32)

    h0 = _embed(x, W_embed, b_embed.reshape(1, H))
    agg1, deg_raw = _msgpass_deg(h0, src_p, dst_p, za, zd)
    deg = deg_raw.reshape(NP, 1)
    h1 = _dense(agg1, deg, W1, b1.reshape(1, H))
    agg2 = _msgpass(h1, src_p, dst_p, za)
    h2, val = _dense_val(agg2, deg, W2, b2.reshape(1, H),
                         W_val, b_val.reshape(1, 1))

    inf_mask = jnp.clip(jnp.log(action_mask[0]), FLOAT_MIN, FLOAT_MAX)
    bias_mask = (b_pol[0] + inf_mask).reshape(D, 1)
    emb = _head(h2, defense_indices[0])
    pol = _pol(emb, W_pol, bias_mask)
    return pol.reshape(1, D), val.reshape(1, 1)


# speedup vs baseline: 38.7845x; 1.1662x over previous
import functools

import jax
import jax.numpy as jnp
import numpy as np
from jax import lax
from jax.experimental import pallas as pl
from jax.experimental.pallas import tpu as pltpu
from jax.experimental.pallas import tpu_sc as plsc

N = 100000
E = 1600000
H = 32
D = 128
FLOAT_MIN = -3.4028235e38
FLOAT_MAX = 3.4028235e38

NC = 2
NS = 16
N2 = N // NC
TRASH = 1200
M = N2 + TRASH
NP = NC * M
PER_TILE = M // NS
WIN = 256
NWIN = 392
E_PAD_T = WIN * NWIN
E_PAD = E_PAD_T * NS

_mesh = plsc.VectorSubcoreMesh(
    core_axis_name="c", subcore_axis_name="s", num_cores=NC, num_subcores=NS
)


def _msgpass_body(with_deg, padded_table, *refs):
    if with_deg:
        (h_hbm, src_hbm, dst_hbm, za_hbm, zd_hbm, agg_hbm, deg_hbm,
         agg_sp, deg_sp,
         src_v0, dst_v0, src2d0, dst2d0, rows0, isem0, gsem0, ssem0,
         src_v1, dst_v1, src2d1, dst2d1, rows1, isem1, gsem1, ssem1,
         ones_v) = refs
    else:
        (h_hbm, src_hbm, dst_hbm, za_hbm, agg_hbm,
         agg_sp,
         src_v0, dst_v0, src2d0, dst2d0, rows0, isem0, gsem0, ssem0,
         src_v1, dst_v1, src2d1, dst2d1, rows1, isem1, gsem1, ssem1,
         ) = refs
        deg_sp = ones_v = None
    bufs = ((src_v0, dst_v0, src2d0, dst2d0, rows0, isem0, gsem0, ssem0),
            (src_v1, dst_v1, src2d1, dst2d1, rows1, isem1, gsem1, ssem1))

    c = lax.axis_index("c")
    s = lax.axis_index("s")
    base = c * N2

    pltpu.sync_copy(za_hbm.at[pl.ds(s * PER_TILE, PER_TILE)],
                    agg_sp.at[pl.ds(s * PER_TILE, PER_TILE)])
    if with_deg:
        pltpu.sync_copy(zd_hbm.at[pl.ds(s * PER_TILE, PER_TILE)],
                        deg_sp.at[pl.ds(s * PER_TILE, PER_TILE)])
        one = jnp.full((16,), 1.0, dtype=jnp.float32)
        for j in range(8):
            ones_v[pl.ds(j * 16, 16)] = one
    plsc.subcore_barrier()

    lanes = lax.broadcasted_iota(jnp.int32, (16,), 0)
    nch = WIN // 128

    def issue_idx(w, b):
        (src_v, dst_v, _, _, _, isem, _, _) = bufs[b]
        e0 = s * E_PAD_T + w * WIN
        return (pltpu.async_copy(src_hbm.at[pl.ds(e0, WIN)], src_v, isem),
                pltpu.async_copy(dst_hbm.at[pl.ds(e0, WIN)], dst_v, isem))

    def stage_and_gather(w, b, idx_cp):
        (src_v, dst_v, src2d, dst2d, rows, _, gsem, _) = bufs[b]
        for cp in idx_cp:
            cp.wait()
        for i in range(WIN // 16):
            sv = src_v[pl.ds(i * 16, 16)]
            if padded_table:
                sv = sv + jnp.where(sv >= N2, TRASH, 0)
            src2d[i // 8, pl.ds((i % 8) * 16, 16)] = sv
        gcopies = []
        for j in range(nch):
            gcopies.append(pltpu.async_copy(
                h_hbm.at[src2d.at[j]], rows.at[pl.ds(j * 128, 128)], gsem))
        for i in range(WIN // 16):
            d = dst_v[pl.ds(i * 16, 16)] - base
            oob = (d < 0) | (d >= N2)
            mix = lanes + (i * 16 + s * 67 + w * 131)
            tr = N2 + (mix & 1023)
            dst2d[i // 8, pl.ds((i % 8) * 16, 16)] = jnp.where(oob, tr, d)
        return gcopies

    def scatter(b, gcopies):
        (_, _, _, dst2d, rows, _, _, ssem) = bufs[b]
        for cp in gcopies:
            cp.wait()
        scopies = []
        for j in range(nch):
            scopies.append(pltpu.async_copy(
                rows.at[pl.ds(j * 128, 128)], agg_sp.at[dst2d.at[j]], ssem,
                add=True))
        if with_deg:
            scopies += [pltpu.async_copy(ones_v, deg_sp.at[dst2d.at[j]],
                                         ssem, add=True) for j in range(nch)]
        return scopies

    def pair(p, carry):
        w0 = p * 2
        i0 = issue_idx(w0, 0)
        i1 = issue_idx(w0 + 1, 1)
        g0 = stage_and_gather(w0, 0, i0)
        g1 = stage_and_gather(w0 + 1, 1, i1)
        s0 = scatter(0, g0)
        s1 = scatter(1, g1)
        for cp in s0 + s1:
            cp.wait()
        return carry

    lax.fori_loop(0, NWIN // 2, pair, 0)
    plsc.subcore_barrier()

    pltpu.sync_copy(agg_sp.at[pl.ds(s * PER_TILE, PER_TILE)],
                    agg_hbm.at[pl.ds(c * M + s * PER_TILE, PER_TILE)])
    if with_deg:
        pltpu.sync_copy(deg_sp.at[pl.ds(s * PER_TILE, PER_TILE)],
                        deg_hbm.at[c, pl.ds(s * PER_TILE, PER_TILE)])


def _make_msgpass(with_deg, padded_table):
    out_type = [jax.ShapeDtypeStruct((NP, H), jnp.float32)]
    scratch = [pltpu.VMEM_SHARED((M, H), jnp.float32)]
    if with_deg:
        out_type.append(jax.ShapeDtypeStruct((NC, M), jnp.float32))
        scratch.append(pltpu.VMEM_SHARED((M,), jnp.float32))
    for _ in range(2):
        scratch += [
            pltpu.VMEM((WIN,), jnp.int32),
            pltpu.VMEM((WIN,), jnp.int32),
            pltpu.VMEM((WIN // 128, 128), jnp.int32),
            pltpu.VMEM((WIN // 128, 128), jnp.int32),
            pltpu.VMEM((WIN, H), jnp.float32),
            pltpu.SemaphoreType.DMA,
            pltpu.SemaphoreType.DMA,
            pltpu.SemaphoreType.DMA,
        ]
    if with_deg:
        scratch.append(pltpu.VMEM((128,), jnp.float32))
    return pl.kernel(
        functools.partial(_msgpass_body, with_deg, padded_table),
        out_type=tuple(out_type) if with_deg else out_type[0],
        mesh=_mesh,
        scratch_types=scratch,
        compiler_params=pltpu.CompilerParams(use_tc_tiling_on_sc=False),
    )


_msgpass_deg = _make_msgpass(True, False)
_msgpass = _make_msgpass(False, True)


def _head_body(h_hbm, didx_hbm, emb_hbm, didx_v, didx_m, rows, gsem):
    c = lax.axis_index("c")
    s = lax.axis_index("s")

    @pl.when((c == 0) & (s == 0))
    def _():
        pltpu.sync_copy(didx_hbm, didx_v)
        for i in range(D // 16):
            dv = didx_v[pl.ds(i * 16, 16)]
            didx_m[pl.ds(i * 16, 16)] = dv + jnp.where(dv >= N2, TRASH, 0)
        pltpu.async_copy(h_hbm.at[didx_m], rows, gsem).wait()
        pltpu.sync_copy(rows, emb_hbm)


_head = pl.kernel(
    _head_body,
    out_type=jax.ShapeDtypeStruct((D, H), jnp.float32),
    mesh=_mesh,
    scratch_types=[
        pltpu.VMEM((D,), jnp.int32),
        pltpu.VMEM((D,), jnp.int32),
        pltpu.VMEM((D, H), jnp.float32),
        pltpu.SemaphoreType.DMA,
    ],
    compiler_params=pltpu.CompilerParams(use_tc_tiling_on_sc=False,
                                         needs_layout_passes=False),
)


def _pol_body(emb_ref, wp_ref, bm_ref, o_ref):
    o_ref[...] = (
        jnp.dot(emb_ref[...], wp_ref[...],
                preferred_element_type=jnp.float32) + bm_ref[...])


_pol = pl.pallas_call(
    _pol_body,
    in_specs=[
        pl.BlockSpec((D, H), lambda: (0, 0)),
        pl.BlockSpec((H, 1), lambda: (0, 0)),
        pl.BlockSpec((D, 1), lambda: (0, 0)),
    ],
    out_specs=pl.BlockSpec((D, 1), lambda: (0, 0)),
    out_shape=jax.ShapeDtypeStruct((D, 1), jnp.float32),
)


BN = 3200
NB = NP // BN


def _embed_body(x_ref, w_ref, b_ref, o_ref):
    o_ref[...] = jnp.tanh(x_ref[...] * w_ref[...] + b_ref[...])


_embed = pl.pallas_call(
    _embed_body,
    grid=(N // 4000,),
    in_specs=[
        pl.BlockSpec((4000, 1), lambda i: (i, 0)),
        pl.BlockSpec((1, H), lambda i: (0, 0)),
        pl.BlockSpec((1, H), lambda i: (0, 0)),
    ],
    out_specs=pl.BlockSpec((4000, H), lambda i: (i, 0)),
    out_shape=jax.ShapeDtypeStruct((N, H), jnp.float32),
)


def _dense_body(agg_ref, deg_ref, w_ref, b_ref, o_ref):
    a = agg_ref[...] / jnp.maximum(deg_ref[...], 1.0)
    o_ref[...] = jnp.maximum(
        jnp.dot(a, w_ref[...], preferred_element_type=jnp.float32)
        + b_ref[...], 0.0)


_dense = pl.pallas_call(
    _dense_body,
    grid=(NB,),
    in_specs=[
        pl.BlockSpec((BN, H), lambda i: (i, 0)),
        pl.BlockSpec((BN, 1), lambda i: (i, 0)),
        pl.BlockSpec((H, H), lambda i: (0, 0)),
        pl.BlockSpec((1, H), lambda i: (0, 0)),
    ],
    out_specs=pl.BlockSpec((BN, H), lambda i: (i, 0)),
    out_shape=jax.ShapeDtypeStruct((NP, H), jnp.float32),
)


def _dense_val_body(agg_ref, deg_ref, w_ref, b_ref, wv_ref, bv_ref,
                    h_ref, val_ref, acc_ref):
    i = pl.program_id(0)
    a = agg_ref[...] / jnp.maximum(deg_ref[...], 1.0)
    hb = jnp.maximum(
        jnp.dot(a, w_ref[...], preferred_element_type=jnp.float32)
        + b_ref[...], 0.0)
    h_ref[...] = hb

    @pl.when(i == 0)
    def _():
        acc_ref[...] = jnp.zeros_like(acc_ref)

    half_blocks = NB // NC
    row_in_half = (i % half_blocks) * BN
    limit = jnp.minimum(jnp.maximum(N2 - row_in_half, 0), BN)
    riota = lax.broadcasted_iota(jnp.int32, (BN, 1), 0)
    hmask = jnp.where(riota < limit, hb, 0.0)
    acc_ref[...] += jnp.sum(hmask, axis=0, keepdims=True)

    @pl.when(i == NB - 1)
    def _():
        val_ref[...] = (
            jnp.dot(acc_ref[...] / np.float32(N), wv_ref[...],
                    preferred_element_type=jnp.float32)
            + bv_ref[...])


_dense_val = pl.pallas_call(
    _dense_val_body,
    grid=(NB,),
    in_specs=[
        pl.BlockSpec((BN, H), lambda i: (i, 0)),
        pl.BlockSpec((BN, 1), lambda i: (i, 0)),
        pl.BlockSpec((H, H), lambda i: (0, 0)),
        pl.BlockSpec((1, H), lambda i: (0, 0)),
        pl.BlockSpec((H, 1), lambda i: (0, 0)),
        pl.BlockSpec((1, 1), lambda i: (0, 0)),
    ],
    out_specs=[
        pl.BlockSpec((BN, H), lambda i: (i, 0)),
        pl.BlockSpec((1, 1), lambda i: (0, 0)),
    ],
    out_shape=[
        jax.ShapeDtypeStruct((NP, H), jnp.float32),
        jax.ShapeDtypeStruct((1, 1), jnp.float32),
    ],
    scratch_shapes=[pltpu.VMEM((1, H), jnp.float32)],
)


def kernel(ids_observation, action_mask, edges, defense_indices, seq_lens,
           W_embed, b_embed, W1, b1, W2, b2, W_pol, b_pol, W_val, b_val):
    x = ids_observation[0].reshape(N, 1)
    src = edges[0, :, 0]
    dst = edges[0, :, 1]
    npad = E_PAD - E
    pad_src = (jnp.arange(npad, dtype=jnp.int32) * 977) % N
    pad_dst = jnp.full((npad,), N, dtype=jnp.int32)
    src_p = jnp.concatenate([src, pad_src])
    dst_p = jnp.concatenate([dst, pad_dst])
    za = jnp.zeros((M, H), jnp.float32)
    zd = jnp.zeros((M,), jnp.float32)

    h0 = _embed(x, W_embed, b_embed.reshape(1, H))
    agg1, deg_raw = _msgpass_deg(h0, src_p, dst_p, za, zd)
    deg = deg_raw.reshape(NP, 1)
    h1 = _dense(agg1, deg, W1, b1.reshape(1, H))
    agg2 = _msgpass(h1, src_p, dst_p, za)
    h2, val = _dense_val(agg2, deg, W2, b2.reshape(1, H),
                         W_val, b_val.reshape(1, 1))

    inf_mask = jnp.clip(jnp.log(action_mask[0]), FLOAT_MIN, FLOAT_MAX)
    bias_mask = (b_pol[0] + inf_mask).reshape(D, 1)
    emb = _head(h2, defense_indices[0])
    pol = _pol(emb, W_pol, bias_mask)
    return pol.reshape(1, D), val.reshape(1, 1)
